# baseline (speedup 1.0000x reference)
import functools

import jax
import jax.numpy as jnp
from jax.experimental import pallas as pl
from jax.experimental.pallas import tpu as pltpu

N = 16384
C = 1000
NUM_PROB_BINS = 10
ALPHA = 0.99
ROWS = 1024
NB = N // ROWS
PB_PAD = 128


def _body(pred_ref, tgt_ref, ce_ref, pb_ref,
          loss_out, pb_out, cls_out,
          loss_acc, hist_acc, cls_acc):
    i = pl.program_id(0)

    @pl.when(i == 0)
    def _init():
        loss_acc[0, 0] = 0.0
        hist_acc[...] = jnp.zeros_like(hist_acc)
        cls_acc[...] = jnp.zeros_like(cls_acc)

    x = pred_ref[...]
    tgt = tgt_ref[...]

    m = jnp.max(x, axis=1, keepdims=True)
    e = jnp.exp(x - m)
    s = jnp.sum(e, axis=1, keepdims=True)

    cls_iota = jax.lax.broadcasted_iota(jnp.int32, (ROWS, C), 1)
    mask = cls_iota == tgt
    t_val = jnp.sum(jnp.where(mask, x, 0.0), axis=1, keepdims=True)
    ce_t = jnp.sum(jnp.where(mask, ce_ref[...], 0.0), axis=1, keepdims=True)

    lse = m + jnp.log(s)
    loss = lse - t_val
    p = jnp.exp(t_val - m) / s
    p = jnp.clip(p, 1e-06, 1.0 - 1e-06)

    bin_idx = jnp.clip(jnp.floor(p * NUM_PROB_BINS - 1e-06).astype(jnp.int32),
                       0, NUM_PROB_BINS - 1)
    lane_iota = jax.lax.broadcasted_iota(jnp.int32, (ROWS, PB_PAD), 1)
    pb_t = jnp.sum(jnp.where(lane_iota == bin_idx, pb_ref[...], 0.0),
                   axis=1, keepdims=True)

    w = jnp.sqrt(ce_t * pb_t + 1e-10)
    loss_acc[0, 0] += jnp.sum(loss / w)

    hist_idx = jnp.clip(jnp.floor(p * NUM_PROB_BINS).astype(jnp.int32),
                        0, NUM_PROB_BINS - 1)
    hist_part = jnp.sum((lane_iota == hist_idx).astype(jnp.float32), axis=0)
    hist_acc[...] += hist_part[None, :]

    cls_acc[...] += jnp.sum(mask.astype(jnp.float32), axis=0)[None, :]

    @pl.when(i == NB - 1)
    def _finalize():
        loss_out[...] = jnp.full((1, PB_PAD), loss_acc[0, 0] / N)

        hist = hist_acc[...]
        prob_bins = hist / (jnp.sum(hist) + 1e-10) * NUM_PROB_BINS
        new_pb = pb_ref[...] * ALPHA + (1.0 - ALPHA) * prob_bins
        new_pb = new_pb / (jnp.sum(new_pb) + 1e-10) * NUM_PROB_BINS
        pb_out[...] = new_pb

        cls = cls_acc[...]
        classes = cls / (jnp.sum(cls) + 1e-10) * C
        new_cls = ce_ref[...] * ALPHA + (1.0 - ALPHA) * classes
        new_cls = new_cls / (jnp.sum(new_cls) + 1e-10) * C
        cls_out[...] = new_cls


@functools.partial(jax.jit, static_argnames=())
def kernel(pred, target, classes_ema, prob_bins_ema):
    tgt2 = target.reshape(N, 1)
    ce2 = classes_ema.reshape(1, C)
    pb2 = jnp.pad(prob_bins_ema, (0, PB_PAD - NUM_PROB_BINS)).reshape(1, PB_PAD)

    loss_o, pb_o, cls_o = pl.pallas_call(
        _body,
        grid=(NB,),
        in_specs=[
            pl.BlockSpec((ROWS, C), lambda i: (i, 0)),
            pl.BlockSpec((ROWS, 1), lambda i: (i, 0)),
            pl.BlockSpec((1, C), lambda i: (0, 0)),
            pl.BlockSpec((1, PB_PAD), lambda i: (0, 0)),
        ],
        out_specs=[
            pl.BlockSpec((1, PB_PAD), lambda i: (0, 0)),
            pl.BlockSpec((1, PB_PAD), lambda i: (0, 0)),
            pl.BlockSpec((1, C), lambda i: (0, 0)),
        ],
        out_shape=[
            jax.ShapeDtypeStruct((1, PB_PAD), jnp.float32),
            jax.ShapeDtypeStruct((1, PB_PAD), jnp.float32),
            jax.ShapeDtypeStruct((1, C), jnp.float32),
        ],
        scratch_shapes=[
            pltpu.SMEM((1, 1), jnp.float32),
            pltpu.VMEM((1, PB_PAD), jnp.float32),
            pltpu.VMEM((1, C), jnp.float32),
        ],
        compiler_params=pltpu.CompilerParams(
            dimension_semantics=("arbitrary",),
        ),
    )(pred, tgt2, ce2, pb2)

    return loss_o[0, 0], pb_o[0, :NUM_PROB_BINS], cls_o[0, :]


# speedup vs baseline: 1.0586x; 1.0586x over previous
import functools

import jax
import jax.numpy as jnp
from jax.experimental import pallas as pl
from jax.experimental.pallas import tpu as pltpu

N = 16384
C = 1000
NUM_PROB_BINS = 10
ALPHA = 0.99
ROWS = 1024
NB = N // ROWS
PB_PAD = 128


def _body(pred_ref, tgt_ref, ce_ref, pb_ref,
          loss_out, pb_out, cls_out,
          loss_acc, hist_acc, cls_acc):
    i = pl.program_id(0)

    @pl.when(i == 0)
    def _init():
        loss_acc[0, 0] = 0.0
        hist_acc[...] = jnp.zeros_like(hist_acc)
        cls_acc[...] = jnp.zeros_like(cls_acc)

    x = pred_ref[...]
    tgt = tgt_ref[...]

    m = jnp.max(x, axis=1, keepdims=True)
    e = jnp.exp(x - m)
    s = jax.lax.dot_general(e, jnp.ones((C, 1), jnp.float32),
                            (((1,), (0,)), ((), ())),
                            preferred_element_type=jnp.float32)

    cls_iota = jax.lax.broadcasted_iota(jnp.int32, (ROWS, C), 1)
    onehot = (cls_iota == tgt).astype(jnp.float32)
    t_val = jnp.sum(x * onehot, axis=1, keepdims=True)
    ce_t = jnp.sum(ce_ref[...] * onehot, axis=1, keepdims=True)

    lse = m + jnp.log(s)
    loss = lse - t_val
    p = jnp.exp(t_val - m) / s
    p = jnp.clip(p, 1e-06, 1.0 - 1e-06)

    bin_idx = jnp.clip(jnp.floor(p * NUM_PROB_BINS - 1e-06).astype(jnp.int32),
                       0, NUM_PROB_BINS - 1)
    lane_iota = jax.lax.broadcasted_iota(jnp.int32, (ROWS, PB_PAD), 1)
    pb_t = jnp.sum(jnp.where(lane_iota == bin_idx, pb_ref[...], 0.0),
                   axis=1, keepdims=True)

    w = jnp.sqrt(ce_t * pb_t + 1e-10)
    loss_acc[0, 0] += jnp.sum(loss / w)

    hist_idx = jnp.clip(jnp.floor(p * NUM_PROB_BINS).astype(jnp.int32),
                        0, NUM_PROB_BINS - 1)
    hist_part = jnp.sum((lane_iota == hist_idx).astype(jnp.float32), axis=0)
    hist_acc[...] += hist_part[None, :]

    cls_acc[...] += jnp.sum(onehot, axis=0)[None, :]

    @pl.when(i == NB - 1)
    def _finalize():
        loss_out[...] = jnp.full((1, PB_PAD), loss_acc[0, 0] / N)

        hist = hist_acc[...]
        prob_bins = hist / (jnp.sum(hist) + 1e-10) * NUM_PROB_BINS
        new_pb = pb_ref[...] * ALPHA + (1.0 - ALPHA) * prob_bins
        new_pb = new_pb / (jnp.sum(new_pb) + 1e-10) * NUM_PROB_BINS
        pb_out[...] = new_pb

        cls = cls_acc[...]
        classes = cls / (jnp.sum(cls) + 1e-10) * C
        new_cls = ce_ref[...] * ALPHA + (1.0 - ALPHA) * classes
        new_cls = new_cls / (jnp.sum(new_cls) + 1e-10) * C
        cls_out[...] = new_cls


@functools.partial(jax.jit, static_argnames=())
def kernel(pred, target, classes_ema, prob_bins_ema):
    tgt2 = target.reshape(N, 1)
    ce2 = classes_ema.reshape(1, C)
    pb2 = jnp.pad(prob_bins_ema, (0, PB_PAD - NUM_PROB_BINS)).reshape(1, PB_PAD)

    loss_o, pb_o, cls_o = pl.pallas_call(
        _body,
        grid=(NB,),
        in_specs=[
            pl.BlockSpec((ROWS, C), lambda i: (i, 0)),
            pl.BlockSpec((ROWS, 1), lambda i: (i, 0)),
            pl.BlockSpec((1, C), lambda i: (0, 0)),
            pl.BlockSpec((1, PB_PAD), lambda i: (0, 0)),
        ],
        out_specs=[
            pl.BlockSpec((1, PB_PAD), lambda i: (0, 0)),
            pl.BlockSpec((1, PB_PAD), lambda i: (0, 0)),
            pl.BlockSpec((1, C), lambda i: (0, 0)),
        ],
        out_shape=[
            jax.ShapeDtypeStruct((1, PB_PAD), jnp.float32),
            jax.ShapeDtypeStruct((1, PB_PAD), jnp.float32),
            jax.ShapeDtypeStruct((1, C), jnp.float32),
        ],
        scratch_shapes=[
            pltpu.SMEM((1, 1), jnp.float32),
            pltpu.VMEM((1, PB_PAD), jnp.float32),
            pltpu.VMEM((1, C), jnp.float32),
        ],
        compiler_params=pltpu.CompilerParams(
            dimension_semantics=("arbitrary",),
        ),
    )(pred, tgt2, ce2, pb2)

    return loss_o[0, 0], pb_o[0, :NUM_PROB_BINS], cls_o[0, :]
